# Initial kernel scaffold; baseline (speedup 1.0000x reference)
import jax
import jax.numpy as jnp
from jax.experimental import pallas as pl


def kernel(ssg_rel, ssg_obj, ssg_att, W_emb, W_sbj, b_sbj, W_objr, b_objr, W_rel, b_rel, W_obj, b_obj, W_att, b_att):
    raise NotImplementedError("write your pallas kernel here")


# speedup vs baseline: 411.1604x; 411.1604x over previous
import functools

import jax
import jax.numpy as jnp
from jax import lax
from jax.experimental import pallas as pl
from jax.experimental.pallas import tpu as pltpu
from jax.experimental.pallas import tpu_sc as plsc

_B = 32
_NREL = 1024
_NOBJ = 512
_NATT = 8
_D = 128

_NW = 32
_TOTAL_ROWS = _B * (_NREL + _NOBJ + _NOBJ * _NATT)
_PER_W = _TOTAL_ROWS // _NW
_CHUNK = 128
_K = 4
_GROUPS = _PER_W // (_CHUNK * _K)


def _sc_gather(table, idx_w):
    mesh = plsc.VectorSubcoreMesh(core_axis_name="c", subcore_axis_name="s")

    @functools.partial(
        pl.kernel,
        out_type=jax.ShapeDtypeStruct((_TOTAL_ROWS, _D), jnp.float32),
        mesh=mesh,
        scratch_types=[
            pltpu.VMEM((_GROUPS * _K, _CHUNK), jnp.int32),
            pltpu.VMEM((_K * _CHUNK, _D), jnp.float32),
            pltpu.SemaphoreType.DMA,
        ],
    )
    def gather_kernel(table_hbm, idx_hbm, out_hbm, idx_v, rows_v, sem):
        wid = lax.axis_index("s") * 2 + lax.axis_index("c")
        base = wid * _PER_W
        pltpu.sync_copy(idx_hbm.at[wid], idx_v)

        @pl.loop(0, _GROUPS)
        def _group(g):
            descs = []
            for j in range(_K):
                descs.append(pltpu.async_copy(
                    table_hbm.at[idx_v.at[g * _K + j]],
                    rows_v.at[pl.ds(j * _CHUNK, _CHUNK)],
                    sem))
            for d in descs:
                d.wait()
            pltpu.sync_copy(
                rows_v,
                out_hbm.at[pl.ds(base + g * (_K * _CHUNK), _K * _CHUNK)])

    return gather_kernel(table, idx_w)


def _tc_body(obj_emb_ref, rel_emb_ref, att_emb_ref,
             sbj_row_ref, obj_row_ref, sbj_col_ref, obj_col_ref,
             keep_rel_ref, keep_obj_ref, m_att_ref, inv_cnt_ref,
             w_sbj_ref, b_sbj_ref, w_objr_ref, b_objr_ref,
             w_rel_ref, b_rel_ref, w_obj_ref, b_obj_ref,
             w_att_ref, b_att_ref, out_ref):
    f32 = jnp.float32
    obj_emb = obj_emb_ref[...]
    rel_emb = rel_emb_ref[...]

    sbj_r = sbj_row_ref[0]
    obj_r = obj_row_ref[0]
    sbj_c = sbj_col_ref[0]
    obj_c = obj_col_ref[0]

    ci = lax.broadcasted_iota(jnp.int32, (_NREL, _NOBJ), 1)
    s_oh = (ci == sbj_c).astype(f32)
    o_oh = (ci == obj_c).astype(f32)
    ct = lax.broadcasted_iota(jnp.int32, (_NOBJ, _NREL), 0)
    s_oh_t = (ct == sbj_r).astype(f32)
    o_oh_t = (ct == obj_r).astype(f32)

    rel_sbj_feat = jnp.dot(s_oh, obj_emb, preferred_element_type=f32)
    rel_obj_feat = jnp.dot(o_oh, obj_emb, preferred_element_type=f32)

    keep_rel = keep_rel_ref[0]

    def mlp3(x1, x2, x3, w_ref, b_ref):
        acc = jnp.dot(x1, w_ref[0:_D], preferred_element_type=f32)
        acc = acc + jnp.dot(x2, w_ref[_D:2 * _D], preferred_element_type=f32)
        acc = acc + jnp.dot(x3, w_ref[2 * _D:3 * _D], preferred_element_type=f32)
        return jnp.maximum(acc + b_ref[...], 0.0)

    rel_sbj_new = mlp3(rel_sbj_feat, rel_obj_feat, rel_emb,
                       w_sbj_ref, b_sbj_ref) * keep_rel
    rel_obj_new = mlp3(rel_sbj_new, rel_obj_feat, rel_emb,
                       w_objr_ref, b_objr_ref) * keep_rel
    rel_rel_feat = mlp3(rel_sbj_new, rel_obj_new, rel_emb,
                        w_rel_ref, b_rel_ref) * keep_rel

    obj_feat = jnp.maximum(
        jnp.dot(obj_emb, w_obj_ref[...], preferred_element_type=f32)
        + b_obj_ref[...], 0.0) * keep_obj_ref[0]
    obj_feat = obj_feat + jnp.dot(s_oh_t, rel_sbj_new, preferred_element_type=f32)
    obj_feat = obj_feat + jnp.dot(o_oh_t, rel_obj_new, preferred_element_type=f32)

    att_emb = att_emb_ref[...]
    t_obj = jnp.dot(obj_emb, w_att_ref[0:_D], preferred_element_type=f32)
    t_att = jnp.dot(att_emb, w_att_ref[_D:2 * _D], preferred_element_type=f32)
    af = jnp.maximum(
        t_att.reshape(_NOBJ, _NATT, _D) + t_obj[:, None, :] + b_att_ref[...],
        0.0)
    af = af * m_att_ref[0].reshape(_NOBJ, _NATT, 1)
    attr_feat = jnp.sum(af, axis=1) * inv_cnt_ref[0]

    out_ref[0, 0:_NOBJ, :] = obj_feat * (1.0 / (2 * _NREL))
    out_ref[0, _NOBJ:_NOBJ + _NREL, :] = rel_rel_feat
    out_ref[0, _NOBJ + _NREL:, :] = attr_feat


def _tc_specs():
    def b3(b):
        return (b, 0, 0)

    def w2(b):
        return (0, 0)

    in_specs = [
        pl.BlockSpec((_NOBJ, _D), lambda b: (_B * _NREL // _NOBJ + b, 0)),
        pl.BlockSpec((_NREL, _D), lambda b: (b, 0)),
        pl.BlockSpec((_NATT * _NOBJ, _D),
                     lambda b: ((_B * _NREL + _B * _NOBJ) // (_NATT * _NOBJ) + b, 0)),
        pl.BlockSpec((1, 1, _NREL), b3),
        pl.BlockSpec((1, 1, _NREL), b3),
        pl.BlockSpec((1, _NREL, 1), b3),
        pl.BlockSpec((1, _NREL, 1), b3),
        pl.BlockSpec((1, _NREL, 1), b3),
        pl.BlockSpec((1, _NOBJ, 1), b3),
        pl.BlockSpec((1, _NATT * _NOBJ, 1), b3),
        pl.BlockSpec((1, _NOBJ, 1), b3),
        pl.BlockSpec((3 * _D, _D), w2),
        pl.BlockSpec((1, _D), w2),
        pl.BlockSpec((3 * _D, _D), w2),
        pl.BlockSpec((1, _D), w2),
        pl.BlockSpec((3 * _D, _D), w2),
        pl.BlockSpec((1, _D), w2),
        pl.BlockSpec((_D, _D), w2),
        pl.BlockSpec((1, _D), w2),
        pl.BlockSpec((2 * _D, _D), w2),
        pl.BlockSpec((1, _D), w2),
    ]
    out_spec = pl.BlockSpec((1, _NOBJ + _NREL + _NOBJ, _D), b3)
    return in_specs, out_spec


def kernel(ssg_rel, ssg_obj, ssg_att, W_emb, W_sbj, b_sbj, W_objr, b_objr,
           W_rel, b_rel, W_obj, b_obj, W_att, b_att):
    f32 = jnp.float32
    sbj_id = ssg_rel[:, :, 0]
    obj_id = ssg_rel[:, :, 1]
    pred_id = ssg_rel[:, :, 2]

    idx_flat = jnp.concatenate(
        [pred_id.reshape(-1), ssg_obj.reshape(-1), ssg_att.reshape(-1)])
    idx_w = idx_flat.reshape(_NW, _GROUPS * _K, _CHUNK)
    gathered = _sc_gather(W_emb, idx_w)

    keep_rel = (pred_id != 1).astype(f32)[:, :, None]
    keep_obj = (ssg_obj != 1).astype(f32)[:, :, None]
    att_ok = ssg_att != 1
    m_att = att_ok.astype(f32).reshape(_B, _NOBJ * _NATT)[:, :, None]
    cnt = jnp.sum(att_ok, axis=-1)
    inv_cnt = jnp.where(cnt == 0, 0.0,
                        1.0 / jnp.maximum(cnt, 1).astype(f32))[:, :, None]

    in_specs, out_spec = _tc_specs()
    ssg_feat = pl.pallas_call(
        _tc_body,
        grid=(_B,),
        in_specs=in_specs,
        out_specs=out_spec,
        out_shape=jax.ShapeDtypeStruct((_B, _NOBJ + _NREL + _NOBJ, _D), f32),
    )(gathered, gathered, gathered,
      sbj_id[:, None, :], obj_id[:, None, :],
      sbj_id[:, :, None], obj_id[:, :, None],
      keep_rel, keep_obj, m_att, inv_cnt,
      W_sbj, b_sbj[None, :], W_objr, b_objr[None, :],
      W_rel, b_rel[None, :], W_obj, b_obj[None, :],
      W_att, b_att[None, :])

    ssg_mask = jnp.concatenate(
        [ssg_obj == 1, pred_id == 1, cnt == 0], axis=1)[:, None, None, :]
    return ssg_feat, ssg_mask


# speedup vs baseline: 491.1200x; 1.1945x over previous
import functools

import jax
import jax.numpy as jnp
from jax import lax
from jax.experimental import pallas as pl
from jax.experimental.pallas import tpu as pltpu
from jax.experimental.pallas import tpu_sc as plsc

_B = 32
_NREL = 1024
_NOBJ = 512
_NATT = 8
_D = 128

_NW = 32
_TOTAL_ROWS = _B * (_NREL + _NOBJ + _NOBJ * _NATT)
_PER_W = _TOTAL_ROWS // _NW
_CHUNK = 128
_K = 4
_GROUPS = _PER_W // (_CHUNK * _K)


def _sc_gather(table, idx_w):
    mesh = plsc.VectorSubcoreMesh(core_axis_name="c", subcore_axis_name="s")

    @functools.partial(
        pl.kernel,
        out_type=jax.ShapeDtypeStruct((_TOTAL_ROWS, _D), jnp.float32),
        mesh=mesh,
        scratch_types=[
            pltpu.VMEM((_GROUPS * _K, _CHUNK), jnp.int32),
            pltpu.VMEM((_K * _CHUNK, _D), jnp.float32),
            pltpu.SemaphoreType.DMA,
        ],
    )
    def gather_kernel(table_hbm, idx_hbm, out_hbm, idx_v, rows_v, sem):
        wid = lax.axis_index("s") * 2 + lax.axis_index("c")
        base = wid * _PER_W
        pltpu.sync_copy(idx_hbm.at[wid], idx_v)

        @pl.loop(0, _GROUPS)
        def _group(g):
            descs = []
            for j in range(_K):
                descs.append(pltpu.async_copy(
                    table_hbm.at[idx_v.at[g * _K + j]],
                    rows_v.at[pl.ds(j * _CHUNK, _CHUNK)],
                    sem))
            for d in descs:
                d.wait()
            pltpu.sync_copy(
                rows_v,
                out_hbm.at[pl.ds(base + g * (_K * _CHUNK), _K * _CHUNK)])

    return gather_kernel(table, idx_w)


def _tc_body(obj_emb_ref, rel_emb_ref, att_emb_ref,
             sbj_row_ref, obj_row_ref, pred_row_ref, objid_row_ref,
             attid_row_ref,
             w_sbj_ref, b_sbj_ref, w_objr_ref, b_objr_ref,
             w_rel_ref, b_rel_ref, w_obj_ref, b_obj_ref,
             w_att_ref, b_att_ref, out_ref):
    f32 = jnp.float32
    obj_emb = obj_emb_ref[...]
    rel_emb = rel_emb_ref[...]

    sbj_r = sbj_row_ref[0]
    obj_r = obj_row_ref[0]
    pred_r = pred_row_ref[0]
    sbj_c = jnp.transpose(sbj_r)
    obj_c = jnp.transpose(obj_r)

    keep_rel = jnp.transpose((pred_r != 1).astype(f32))
    keep_obj = jnp.transpose((objid_row_ref[0] != 1).astype(f32))

    ci = lax.broadcasted_iota(jnp.int32, (_NREL, _NOBJ), 1)
    s_oh = (ci == sbj_c).astype(f32)
    o_oh = (ci == obj_c).astype(f32)
    ct = lax.broadcasted_iota(jnp.int32, (_NOBJ, _NREL), 0)
    s_oh_t = (ct == sbj_r).astype(f32)
    o_oh_t = (ct == obj_r).astype(f32)

    rel_sbj_feat = jnp.dot(s_oh, obj_emb, preferred_element_type=f32)
    rel_obj_feat = jnp.dot(o_oh, obj_emb, preferred_element_type=f32)

    def mlp3(x1, x2, x3, w_ref, b_ref):
        acc = jnp.dot(x1, w_ref[0:_D], preferred_element_type=f32)
        acc = acc + jnp.dot(x2, w_ref[_D:2 * _D], preferred_element_type=f32)
        acc = acc + jnp.dot(x3, w_ref[2 * _D:3 * _D], preferred_element_type=f32)
        return jnp.maximum(acc + b_ref[...], 0.0)

    rel_sbj_new = mlp3(rel_sbj_feat, rel_obj_feat, rel_emb,
                       w_sbj_ref, b_sbj_ref) * keep_rel
    rel_obj_new = mlp3(rel_sbj_new, rel_obj_feat, rel_emb,
                       w_objr_ref, b_objr_ref) * keep_rel
    rel_rel_feat = mlp3(rel_sbj_new, rel_obj_new, rel_emb,
                        w_rel_ref, b_rel_ref) * keep_rel

    obj_feat = jnp.maximum(
        jnp.dot(obj_emb, w_obj_ref[...], preferred_element_type=f32)
        + b_obj_ref[...], 0.0) * keep_obj
    obj_feat = obj_feat + jnp.dot(s_oh_t, rel_sbj_new, preferred_element_type=f32)
    obj_feat = obj_feat + jnp.dot(o_oh_t, rel_obj_new, preferred_element_type=f32)

    att_emb = att_emb_ref[...]
    m_col = jnp.transpose((attid_row_ref[0] != 1).astype(f32))
    m_col = m_col.reshape(_NOBJ, _NATT, 1)
    cnt = jnp.sum(m_col, axis=1)
    inv_cnt = jnp.where(cnt == 0.0, 0.0,
                        1.0 / jnp.where(cnt == 0.0, 1.0, cnt))

    t_obj = jnp.dot(obj_emb, w_att_ref[0:_D], preferred_element_type=f32)
    t_att = jnp.dot(att_emb, w_att_ref[_D:2 * _D], preferred_element_type=f32)
    af = jnp.maximum(
        t_att.reshape(_NOBJ, _NATT, _D) + t_obj[:, None, :] + b_att_ref[...],
        0.0)
    af = af * m_col
    attr_feat = jnp.sum(af, axis=1) * inv_cnt

    out_ref[0, 0:_NOBJ, :] = obj_feat * (1.0 / (2 * _NREL))
    out_ref[0, _NOBJ:_NOBJ + _NREL, :] = rel_rel_feat
    out_ref[0, _NOBJ + _NREL:, :] = attr_feat


def _tc_specs():
    def b3(b):
        return (b, 0, 0)

    def w2(b):
        return (0, 0)

    in_specs = [
        pl.BlockSpec((_NOBJ, _D), lambda b: (_B * _NREL // _NOBJ + b, 0)),
        pl.BlockSpec((_NREL, _D), lambda b: (b, 0)),
        pl.BlockSpec((_NATT * _NOBJ, _D),
                     lambda b: ((_B * _NREL + _B * _NOBJ) // (_NATT * _NOBJ) + b, 0)),
        pl.BlockSpec((1, 1, _NREL), b3),
        pl.BlockSpec((1, 1, _NREL), b3),
        pl.BlockSpec((1, 1, _NREL), b3),
        pl.BlockSpec((1, 1, _NOBJ), b3),
        pl.BlockSpec((1, 1, _NATT * _NOBJ), b3),
        pl.BlockSpec((3 * _D, _D), w2),
        pl.BlockSpec((1, _D), w2),
        pl.BlockSpec((3 * _D, _D), w2),
        pl.BlockSpec((1, _D), w2),
        pl.BlockSpec((3 * _D, _D), w2),
        pl.BlockSpec((1, _D), w2),
        pl.BlockSpec((_D, _D), w2),
        pl.BlockSpec((1, _D), w2),
        pl.BlockSpec((2 * _D, _D), w2),
        pl.BlockSpec((1, _D), w2),
    ]
    out_spec = pl.BlockSpec((1, _NOBJ + _NREL + _NOBJ, _D), b3)
    return in_specs, out_spec


def kernel(ssg_rel, ssg_obj, ssg_att, W_emb, W_sbj, b_sbj, W_objr, b_objr,
           W_rel, b_rel, W_obj, b_obj, W_att, b_att):
    f32 = jnp.float32
    sbj_id = ssg_rel[:, :, 0]
    obj_id = ssg_rel[:, :, 1]
    pred_id = ssg_rel[:, :, 2]

    idx_flat = jnp.concatenate(
        [pred_id.reshape(-1), ssg_obj.reshape(-1), ssg_att.reshape(-1)])
    idx_w = idx_flat.reshape(_NW, _GROUPS * _K, _CHUNK)
    gathered = _sc_gather(W_emb, idx_w)

    in_specs, out_spec = _tc_specs()
    ssg_feat = pl.pallas_call(
        _tc_body,
        grid=(_B,),
        in_specs=in_specs,
        out_specs=out_spec,
        out_shape=jax.ShapeDtypeStruct((_B, _NOBJ + _NREL + _NOBJ, _D), f32),
    )(gathered, gathered, gathered,
      sbj_id[:, None, :], obj_id[:, None, :], pred_id[:, None, :],
      ssg_obj[:, None, :], ssg_att.reshape(_B, 1, _NOBJ * _NATT),
      W_sbj, b_sbj[None, :], W_objr, b_objr[None, :],
      W_rel, b_rel[None, :], W_obj, b_obj[None, :],
      W_att, b_att[None, :])

    cnt = jnp.sum(ssg_att != 1, axis=-1)
    ssg_mask = jnp.concatenate(
        [ssg_obj == 1, pred_id == 1, cnt == 0], axis=1)[:, None, None, :]
    return ssg_feat, ssg_mask


# speedup vs baseline: 552.7864x; 1.1256x over previous
import functools

import jax
import jax.numpy as jnp
from jax import lax
from jax.experimental import pallas as pl
from jax.experimental.pallas import tpu as pltpu
from jax.experimental.pallas import tpu_sc as plsc

_B = 32
_NREL = 1024
_NOBJ = 512
_NATT = 8
_D = 128

_NW = 32
_CHUNK = 128
_K = 4

_ROWS_A = _B * (_NREL + _NOBJ)
_ROWS_B = _B * _NOBJ * _NATT


def _sc_gather(table, idx_w, nchunks):
    per_w = nchunks * _CHUNK
    groups = nchunks // _K
    mesh = plsc.VectorSubcoreMesh(core_axis_name="c", subcore_axis_name="s")

    @functools.partial(
        pl.kernel,
        out_type=jax.ShapeDtypeStruct((_NW * per_w, _D), jnp.float32),
        mesh=mesh,
        scratch_types=[
            pltpu.VMEM((nchunks, _CHUNK), jnp.int32),
            pltpu.VMEM((_K * _CHUNK, _D), jnp.float32),
            pltpu.SemaphoreType.DMA,
        ],
    )
    def gather_kernel(table_hbm, idx_hbm, out_hbm, idx_v, rows_v, sem):
        wid = lax.axis_index("s") * 2 + lax.axis_index("c")
        base = wid * per_w
        pltpu.sync_copy(idx_hbm.at[wid], idx_v)

        @pl.loop(0, groups)
        def _group(g):
            descs = []
            for j in range(_K):
                descs.append(pltpu.async_copy(
                    table_hbm.at[idx_v.at[g * _K + j]],
                    rows_v.at[pl.ds(j * _CHUNK, _CHUNK)],
                    sem))
            for d in descs:
                d.wait()
            pltpu.sync_copy(
                rows_v,
                out_hbm.at[pl.ds(base + g * (_K * _CHUNK), _K * _CHUNK)])

    return gather_kernel(table, idx_w)


def _tc_a_body(obj_emb_ref, rel_emb_ref,
               sbj_row_ref, obj_row_ref, pred_row_ref, objid_row_ref,
               w_sbj_ref, b_sbj_ref, w_objr_ref, b_objr_ref,
               w_rel_ref, b_rel_ref, w_obj_ref, b_obj_ref, out_ref):
    f32 = jnp.float32
    obj_emb = obj_emb_ref[...]
    rel_emb = rel_emb_ref[...]

    sbj_r = sbj_row_ref[0]
    obj_r = obj_row_ref[0]
    pred_r = pred_row_ref[0]

    keep_rel = jnp.transpose((pred_r != 1).astype(f32))
    keep_obj = jnp.transpose((objid_row_ref[0] != 1).astype(f32))

    ct = lax.broadcasted_iota(jnp.int32, (_NOBJ, _NREL), 0)
    s_oh_t = (ct == sbj_r).astype(f32)
    o_oh_t = (ct == obj_r).astype(f32)

    dn_t = (((0,), (0,)), ((), ()))
    rel_sbj_feat = lax.dot_general(s_oh_t, obj_emb, dn_t,
                                   preferred_element_type=f32)
    rel_obj_feat = lax.dot_general(o_oh_t, obj_emb, dn_t,
                                   preferred_element_type=f32)

    def mlp3(x1, x2, x3, w_ref, b_ref):
        acc = jnp.dot(x1, w_ref[0:_D], preferred_element_type=f32)
        acc = acc + jnp.dot(x2, w_ref[_D:2 * _D], preferred_element_type=f32)
        acc = acc + jnp.dot(x3, w_ref[2 * _D:3 * _D], preferred_element_type=f32)
        return jnp.maximum(acc + b_ref[...], 0.0)

    rel_sbj_new = mlp3(rel_sbj_feat, rel_obj_feat, rel_emb,
                       w_sbj_ref, b_sbj_ref) * keep_rel
    rel_obj_new = mlp3(rel_sbj_new, rel_obj_feat, rel_emb,
                       w_objr_ref, b_objr_ref) * keep_rel
    rel_rel_feat = mlp3(rel_sbj_new, rel_obj_new, rel_emb,
                        w_rel_ref, b_rel_ref) * keep_rel

    obj_feat = jnp.maximum(
        jnp.dot(obj_emb, w_obj_ref[...], preferred_element_type=f32)
        + b_obj_ref[...], 0.0) * keep_obj
    obj_feat = obj_feat + jnp.dot(s_oh_t, rel_sbj_new, preferred_element_type=f32)
    obj_feat = obj_feat + jnp.dot(o_oh_t, rel_obj_new, preferred_element_type=f32)

    out_ref[0, 0:_NOBJ, :] = obj_feat * (1.0 / (2 * _NREL))
    out_ref[0, _NOBJ:_NOBJ + _NREL, :] = rel_rel_feat


def _tc_b_body(alias_ref, obj_emb_ref, att_emb_ref, attid_row_ref,
               w_att_ref, b_att_ref, out_ref):
    del alias_ref
    f32 = jnp.float32
    obj_emb = obj_emb_ref[...]
    att_emb = att_emb_ref[...]

    m_col = jnp.transpose((attid_row_ref[0] != 1).astype(f32))

    tb = jnp.dot(obj_emb, w_att_ref[0:_D],
                 preferred_element_type=f32) + b_att_ref[...]
    t_att = jnp.dot(att_emb, w_att_ref[_D:2 * _D],
                    preferred_element_type=f32)

    acc = jnp.zeros((_NOBJ, _D), f32)
    cnt = jnp.zeros((_NOBJ, 1), f32)
    for j in range(_NATT):
        sl = t_att[j * _NOBJ:(j + 1) * _NOBJ, :]
        mj = m_col[j * _NOBJ:(j + 1) * _NOBJ, :]
        acc = acc + jnp.maximum(sl + tb, 0.0) * mj
        cnt = cnt + mj
    inv_cnt = jnp.where(cnt == 0.0, 0.0,
                        1.0 / jnp.where(cnt == 0.0, 1.0, cnt))
    out_ref[0, :, :] = acc * inv_cnt


def kernel(ssg_rel, ssg_obj, ssg_att, W_emb, W_sbj, b_sbj, W_objr, b_objr,
           W_rel, b_rel, W_obj, b_obj, W_att, b_att):
    f32 = jnp.float32
    sbj_id = ssg_rel[:, :, 0]
    obj_id = ssg_rel[:, :, 1]
    pred_id = ssg_rel[:, :, 2]

    idx_a = jnp.concatenate([pred_id.reshape(-1), ssg_obj.reshape(-1)])
    ga = _sc_gather(W_emb, idx_a.reshape(_NW, -1, _CHUNK),
                    _ROWS_A // (_NW * _CHUNK))
    att_t = jnp.transpose(ssg_att, (0, 2, 1))
    idx_b = att_t.reshape(-1)
    gb = _sc_gather(W_emb, idx_b.reshape(_NW, -1, _CHUNK),
                    _ROWS_B // (_NW * _CHUNK))

    def b3(b):
        return (b, 0, 0)

    def w2(b):
        return (0, 0)

    out_shape = jax.ShapeDtypeStruct((_B, _NOBJ + _NREL + _NOBJ, _D), f32)

    feat_a = pl.pallas_call(
        _tc_a_body,
        grid=(_B,),
        in_specs=[
            pl.BlockSpec((_NOBJ, _D), lambda b: (_B * _NREL // _NOBJ + b, 0)),
            pl.BlockSpec((_NREL, _D), lambda b: (b, 0)),
            pl.BlockSpec((1, 1, _NREL), b3),
            pl.BlockSpec((1, 1, _NREL), b3),
            pl.BlockSpec((1, 1, _NREL), b3),
            pl.BlockSpec((1, 1, _NOBJ), b3),
            pl.BlockSpec((3 * _D, _D), w2),
            pl.BlockSpec((1, _D), w2),
            pl.BlockSpec((3 * _D, _D), w2),
            pl.BlockSpec((1, _D), w2),
            pl.BlockSpec((3 * _D, _D), w2),
            pl.BlockSpec((1, _D), w2),
            pl.BlockSpec((_D, _D), w2),
            pl.BlockSpec((1, _D), w2),
        ],
        out_specs=pl.BlockSpec((1, _NOBJ + _NREL + _NOBJ, _D), b3),
        out_shape=out_shape,
    )(ga, ga,
      sbj_id[:, None, :], obj_id[:, None, :], pred_id[:, None, :],
      ssg_obj[:, None, :],
      W_sbj, b_sbj[None, :], W_objr, b_objr[None, :],
      W_rel, b_rel[None, :], W_obj, b_obj[None, :])

    ssg_feat = pl.pallas_call(
        _tc_b_body,
        grid=(_B,),
        in_specs=[
            pl.BlockSpec(memory_space=pl.ANY),
            pl.BlockSpec((_NOBJ, _D), lambda b: (_B * _NREL // _NOBJ + b, 0)),
            pl.BlockSpec((_NATT * _NOBJ, _D), lambda b: (b, 0)),
            pl.BlockSpec((1, 1, _NATT * _NOBJ), b3),
            pl.BlockSpec((2 * _D, _D), w2),
            pl.BlockSpec((1, _D), w2),
        ],
        out_specs=pl.BlockSpec((1, _NOBJ, _D), lambda b: (b, 3, 0)),
        out_shape=out_shape,
        input_output_aliases={0: 0},
    )(feat_a, ga, gb, att_t.reshape(_B, 1, _NATT * _NOBJ),
      W_att, b_att[None, :])

    cnt = jnp.sum(ssg_att != 1, axis=-1)
    ssg_mask = jnp.concatenate(
        [ssg_obj == 1, pred_id == 1, cnt == 0], axis=1)[:, None, None, :]
    return ssg_feat, ssg_mask


# speedup vs baseline: 572.7015x; 1.0360x over previous
import functools

import jax
import jax.numpy as jnp
from jax import lax
from jax.experimental import pallas as pl
from jax.experimental.pallas import tpu as pltpu
from jax.experimental.pallas import tpu_sc as plsc

_B = 32
_NREL = 1024
_NOBJ = 512
_NATT = 8
_D = 128

_NW = 32
_CHUNK = 128
_K = 4

_ROWS_A = _B * (_NREL + _NOBJ)
_ROWS_B = _B * _NOBJ * _NATT


def _sc_gather(table, idx_w, nchunks):
    per_w = nchunks * _CHUNK
    groups = nchunks // _K
    mesh = plsc.VectorSubcoreMesh(core_axis_name="c", subcore_axis_name="s")

    @functools.partial(
        pl.kernel,
        out_type=jax.ShapeDtypeStruct((_NW * per_w, _D), jnp.float32),
        mesh=mesh,
        scratch_types=[
            pltpu.VMEM((nchunks, _CHUNK), jnp.int32),
            pltpu.VMEM((_K * _CHUNK, _D), jnp.float32),
            pltpu.SemaphoreType.DMA,
        ],
    )
    def gather_kernel(table_hbm, idx_hbm, out_hbm, idx_v, rows_v, sem):
        wid = lax.axis_index("s") * 2 + lax.axis_index("c")
        base = wid * per_w
        pltpu.sync_copy(idx_hbm.at[wid], idx_v)

        @pl.loop(0, groups)
        def _group(g):
            descs = []
            for j in range(_K):
                descs.append(pltpu.async_copy(
                    table_hbm.at[idx_v.at[g * _K + j]],
                    rows_v.at[pl.ds(j * _CHUNK, _CHUNK)],
                    sem))
            for d in descs:
                d.wait()
            pltpu.sync_copy(
                rows_v,
                out_hbm.at[pl.ds(base + g * (_K * _CHUNK), _K * _CHUNK)])

    return gather_kernel(table, idx_w)


def _tc_a_body(obj_emb_ref, rel_emb_ref,
               sbj_row_ref, obj_row_ref, pred_row_ref, objid_row_ref,
               w_sbj_ref, b_sbj_ref, w_objr_ref, b_objr_ref,
               w_rel_ref, b_rel_ref, w_obj_ref, b_obj_ref, out_ref):
    f32 = jnp.float32
    obj_emb = obj_emb_ref[...]
    rel_emb = rel_emb_ref[...]

    sbj_r = sbj_row_ref[0]
    obj_r = obj_row_ref[0]
    pred_r = pred_row_ref[0]

    keep_rel = jnp.transpose((pred_r != 1).astype(f32))
    keep_obj = jnp.transpose((objid_row_ref[0] != 1).astype(f32))

    ct = lax.broadcasted_iota(jnp.int32, (_NOBJ, _NREL), 0)
    s_oh_t = (ct == sbj_r).astype(f32)
    o_oh_t = (ct == obj_r).astype(f32)

    dn_t = (((0,), (0,)), ((), ()))
    rel_sbj_feat = lax.dot_general(s_oh_t, obj_emb, dn_t,
                                   preferred_element_type=f32)
    rel_obj_feat = lax.dot_general(o_oh_t, obj_emb, dn_t,
                                   preferred_element_type=f32)

    def mlp3(x1, x2, x3, w_ref, b_ref):
        acc = jnp.dot(x1, w_ref[0:_D], preferred_element_type=f32)
        acc = acc + jnp.dot(x2, w_ref[_D:2 * _D], preferred_element_type=f32)
        acc = acc + jnp.dot(x3, w_ref[2 * _D:3 * _D], preferred_element_type=f32)
        return jnp.maximum(acc + b_ref[...], 0.0)

    rel_sbj_new = mlp3(rel_sbj_feat, rel_obj_feat, rel_emb,
                       w_sbj_ref, b_sbj_ref) * keep_rel
    rel_obj_new = mlp3(rel_sbj_new, rel_obj_feat, rel_emb,
                       w_objr_ref, b_objr_ref) * keep_rel
    rel_rel_feat = mlp3(rel_sbj_new, rel_obj_new, rel_emb,
                        w_rel_ref, b_rel_ref) * keep_rel

    obj_feat = jnp.maximum(
        jnp.dot(obj_emb, w_obj_ref[...], preferred_element_type=f32)
        + b_obj_ref[...], 0.0) * keep_obj
    obj_feat = obj_feat + jnp.dot(s_oh_t, rel_sbj_new, preferred_element_type=f32)
    obj_feat = obj_feat + jnp.dot(o_oh_t, rel_obj_new, preferred_element_type=f32)

    out_ref[0, 0:_NOBJ, :] = obj_feat * (1.0 / (2 * _NREL))
    out_ref[0, _NOBJ:_NOBJ + _NREL, :] = rel_rel_feat


def _tc_b_body(alias_ref, obj_emb_ref, att_emb_ref, attid_ref,
               w_att_ref, b_att_ref, out_ref):
    del alias_ref
    f32 = jnp.float32
    obj_emb = obj_emb_ref[...]
    att_emb = att_emb_ref[...]

    m8 = (attid_ref[0] != 1).astype(f32)
    cnt_row = jnp.sum(m8, axis=0, keepdims=True)
    inv_row = jnp.where(cnt_row == 0.0, 0.0,
                        1.0 / jnp.where(cnt_row == 0.0, 1.0, cnt_row))
    inv_mat = jnp.dot(jnp.transpose(inv_row), jnp.ones((1, _D), f32),
                      preferred_element_type=f32)
    m8t = jnp.transpose(m8)
    bd_r = lax.broadcasted_iota(jnp.int32, (_NATT, _NATT * _D), 0)
    bd_c = lax.broadcasted_iota(jnp.int32, (_NATT, _NATT * _D), 1)
    blockdiag = (bd_r == bd_c // _D).astype(f32)
    m_mat = jnp.dot(m8t, blockdiag,
                    preferred_element_type=f32)

    tb = jnp.dot(obj_emb, w_att_ref[0:_D],
                 preferred_element_type=f32) + b_att_ref[...]
    t_att = jnp.dot(att_emb, w_att_ref[_D:2 * _D],
                    preferred_element_type=f32)

    acc = jnp.zeros((_NOBJ, _D), f32)
    for j in range(_NATT):
        sl = t_att[j * _NOBJ:(j + 1) * _NOBJ, :]
        mj = m_mat[:, j * _D:(j + 1) * _D]
        acc = acc + jnp.maximum(sl + tb, 0.0) * mj
    out_ref[0, :, :] = acc * inv_mat


def kernel(ssg_rel, ssg_obj, ssg_att, W_emb, W_sbj, b_sbj, W_objr, b_objr,
           W_rel, b_rel, W_obj, b_obj, W_att, b_att):
    f32 = jnp.float32
    sbj_id = ssg_rel[:, :, 0]
    obj_id = ssg_rel[:, :, 1]
    pred_id = ssg_rel[:, :, 2]

    idx_a = jnp.concatenate([pred_id.reshape(-1), ssg_obj.reshape(-1)])
    ga = _sc_gather(W_emb, idx_a.reshape(_NW, -1, _CHUNK),
                    _ROWS_A // (_NW * _CHUNK))
    att_t = jnp.transpose(ssg_att, (0, 2, 1))
    idx_b = att_t.reshape(-1)
    gb = _sc_gather(W_emb, idx_b.reshape(_NW, -1, _CHUNK),
                    _ROWS_B // (_NW * _CHUNK))

    def b3(b):
        return (b, 0, 0)

    def w2(b):
        return (0, 0)

    out_shape = jax.ShapeDtypeStruct((_B, _NOBJ + _NREL + _NOBJ, _D), f32)

    feat_a = pl.pallas_call(
        _tc_a_body,
        grid=(_B,),
        in_specs=[
            pl.BlockSpec((_NOBJ, _D), lambda b: (_B * _NREL // _NOBJ + b, 0)),
            pl.BlockSpec((_NREL, _D), lambda b: (b, 0)),
            pl.BlockSpec((1, 1, _NREL), b3),
            pl.BlockSpec((1, 1, _NREL), b3),
            pl.BlockSpec((1, 1, _NREL), b3),
            pl.BlockSpec((1, 1, _NOBJ), b3),
            pl.BlockSpec((3 * _D, _D), w2),
            pl.BlockSpec((1, _D), w2),
            pl.BlockSpec((3 * _D, _D), w2),
            pl.BlockSpec((1, _D), w2),
            pl.BlockSpec((3 * _D, _D), w2),
            pl.BlockSpec((1, _D), w2),
            pl.BlockSpec((_D, _D), w2),
            pl.BlockSpec((1, _D), w2),
        ],
        out_specs=pl.BlockSpec((1, _NOBJ + _NREL + _NOBJ, _D), b3),
        out_shape=out_shape,
    )(ga, ga,
      sbj_id[:, None, :], obj_id[:, None, :], pred_id[:, None, :],
      ssg_obj[:, None, :],
      W_sbj, b_sbj[None, :], W_objr, b_objr[None, :],
      W_rel, b_rel[None, :], W_obj, b_obj[None, :])

    ssg_feat = pl.pallas_call(
        _tc_b_body,
        grid=(_B,),
        in_specs=[
            pl.BlockSpec(memory_space=pl.ANY),
            pl.BlockSpec((_NOBJ, _D), lambda b: (_B * _NREL // _NOBJ + b, 0)),
            pl.BlockSpec((_NATT * _NOBJ, _D), lambda b: (b, 0)),
            pl.BlockSpec((1, _NATT, _NOBJ), b3),
            pl.BlockSpec((2 * _D, _D), w2),
            pl.BlockSpec((1, _D), w2),
        ],
        out_specs=pl.BlockSpec((1, _NOBJ, _D), lambda b: (b, 3, 0)),
        out_shape=out_shape,
        input_output_aliases={0: 0},
    )(feat_a, ga, gb, att_t, W_att, b_att[None, :])

    cnt = jnp.sum(ssg_att != 1, axis=-1)
    ssg_mask = jnp.concatenate(
        [ssg_obj == 1, pred_id == 1, cnt == 0], axis=1)[:, None, None, :]
    return ssg_feat, ssg_mask


# speedup vs baseline: 573.5670x; 1.0015x over previous
import functools

import jax
import jax.numpy as jnp
from jax import lax
from jax.experimental import pallas as pl
from jax.experimental.pallas import tpu as pltpu
from jax.experimental.pallas import tpu_sc as plsc

_B = 32
_NREL = 1024
_NOBJ = 512
_NATT = 8
_D = 128

_NW = 32
_CHUNK = 128
_K = 4

_ROWS_A = _B * (_NREL + _NOBJ)
_ROWS_B = _B * _NOBJ * _NATT


_GC = 2


def _sc_gather(table, idx_w, nchunks):
    per_w = nchunks * _CHUNK
    groups = nchunks // _GC
    grp_rows = _GC * _CHUNK
    mesh = plsc.VectorSubcoreMesh(core_axis_name="c", subcore_axis_name="s")

    @functools.partial(
        pl.kernel,
        out_type=jax.ShapeDtypeStruct((_NW * per_w, _D), jnp.float32),
        mesh=mesh,
        scratch_types=[
            pltpu.VMEM((nchunks, _CHUNK), jnp.int32),
            pltpu.VMEM((grp_rows, _D), jnp.float32),
            pltpu.VMEM((grp_rows, _D), jnp.float32),
            pltpu.SemaphoreType.DMA,
            pltpu.SemaphoreType.DMA,
            pltpu.SemaphoreType.DMA,
            pltpu.SemaphoreType.DMA,
        ],
    )
    def gather_kernel(table_hbm, idx_hbm, out_hbm, idx_v,
                      rows_a, rows_b, gsa, gsb, wsa, wsb):
        wid = lax.axis_index("s") * 2 + lax.axis_index("c")
        base = wid * per_w
        pltpu.sync_copy(idx_hbm.at[wid], idx_v)

        def fire_g(buf, gsem, grp):
            return [pltpu.async_copy(
                table_hbm.at[idx_v.at[grp * _GC + j]],
                buf.at[pl.ds(j * _CHUNK, _CHUNK)], gsem)
                for j in range(_GC)]

        def fire_w(buf, wsem, grp):
            return pltpu.async_copy(
                buf, out_hbm.at[pl.ds(base + grp * grp_rows, grp_rows)], wsem)

        def drain(buf, sem):
            pltpu.make_async_copy(
                out_hbm.at[pl.ds(base, grp_rows)], buf, sem).wait()

        fire_g(rows_a, gsa, 0)

        @pl.loop(0, groups, step=2)
        def _pair(k):
            drain(rows_a, gsa)
            wa = fire_w(rows_a, wsa, k)

            @pl.when(k > 0)
            def _():
                drain(rows_b, wsb)

            gb = fire_g(rows_b, gsb, k + 1)

            for d in gb:
                d.wait()
            fire_w(rows_b, wsb, k + 1)

            @pl.when(k + 2 < groups)
            def _():
                wa.wait()
                fire_g(rows_a, gsa, k + 2)

        drain(rows_a, wsa)
        drain(rows_b, wsb)

    return gather_kernel(table, idx_w)


def _tc_a_body(obj_emb_ref, rel_emb_ref,
               sbj_row_ref, obj_row_ref, pred_row_ref, objid_row_ref,
               w_sbj_ref, b_sbj_ref, w_objr_ref, b_objr_ref,
               w_rel_ref, b_rel_ref, w_obj_ref, b_obj_ref, out_ref):
    f32 = jnp.float32
    obj_emb = obj_emb_ref[...]
    rel_emb = rel_emb_ref[...]

    sbj_r = sbj_row_ref[0]
    obj_r = obj_row_ref[0]
    pred_r = pred_row_ref[0]

    keep_rel = jnp.transpose((pred_r != 1).astype(f32))
    keep_obj = jnp.transpose((objid_row_ref[0] != 1).astype(f32))

    ct = lax.broadcasted_iota(jnp.int32, (_NOBJ, _NREL), 0)
    s_oh_t = (ct == sbj_r).astype(f32)
    o_oh_t = (ct == obj_r).astype(f32)

    dn_t = (((0,), (0,)), ((), ()))
    rel_sbj_feat = lax.dot_general(s_oh_t, obj_emb, dn_t,
                                   preferred_element_type=f32)
    rel_obj_feat = lax.dot_general(o_oh_t, obj_emb, dn_t,
                                   preferred_element_type=f32)

    def mlp3(x1, x2, x3, w_ref, b_ref):
        acc = jnp.dot(x1, w_ref[0:_D], preferred_element_type=f32)
        acc = acc + jnp.dot(x2, w_ref[_D:2 * _D], preferred_element_type=f32)
        acc = acc + jnp.dot(x3, w_ref[2 * _D:3 * _D], preferred_element_type=f32)
        return jnp.maximum(acc + b_ref[...], 0.0)

    rel_sbj_new = mlp3(rel_sbj_feat, rel_obj_feat, rel_emb,
                       w_sbj_ref, b_sbj_ref) * keep_rel
    rel_obj_new = mlp3(rel_sbj_new, rel_obj_feat, rel_emb,
                       w_objr_ref, b_objr_ref) * keep_rel
    rel_rel_feat = mlp3(rel_sbj_new, rel_obj_new, rel_emb,
                        w_rel_ref, b_rel_ref) * keep_rel

    obj_feat = jnp.maximum(
        jnp.dot(obj_emb, w_obj_ref[...], preferred_element_type=f32)
        + b_obj_ref[...], 0.0) * keep_obj
    obj_feat = obj_feat + jnp.dot(s_oh_t, rel_sbj_new, preferred_element_type=f32)
    obj_feat = obj_feat + jnp.dot(o_oh_t, rel_obj_new, preferred_element_type=f32)

    out_ref[0, 0:_NOBJ, :] = obj_feat * (1.0 / (2 * _NREL))
    out_ref[0, _NOBJ:_NOBJ + _NREL, :] = rel_rel_feat


def _tc_b_body(alias_ref, obj_emb_ref, att_emb_ref, attid_ref,
               w_att_ref, b_att_ref, out_ref):
    del alias_ref
    f32 = jnp.float32
    obj_emb = obj_emb_ref[...]
    att_emb = att_emb_ref[...]

    m8 = (attid_ref[0] != 1).astype(f32)
    cnt_row = jnp.sum(m8, axis=0, keepdims=True)
    inv_row = jnp.where(cnt_row == 0.0, 0.0,
                        1.0 / jnp.where(cnt_row == 0.0, 1.0, cnt_row))
    inv_mat = jnp.dot(jnp.transpose(inv_row), jnp.ones((1, _D), f32),
                      preferred_element_type=f32)
    m8t = jnp.transpose(m8)
    bd_r = lax.broadcasted_iota(jnp.int32, (_NATT, _NATT * _D), 0)
    bd_c = lax.broadcasted_iota(jnp.int32, (_NATT, _NATT * _D), 1)
    blockdiag = (bd_r == bd_c // _D).astype(f32)
    m_mat = jnp.dot(m8t, blockdiag,
                    preferred_element_type=f32)

    tb = jnp.dot(obj_emb, w_att_ref[0:_D],
                 preferred_element_type=f32) + b_att_ref[...]
    t_att = jnp.dot(att_emb, w_att_ref[_D:2 * _D],
                    preferred_element_type=f32)

    acc = jnp.zeros((_NOBJ, _D), f32)
    for j in range(_NATT):
        sl = t_att[j * _NOBJ:(j + 1) * _NOBJ, :]
        mj = m_mat[:, j * _D:(j + 1) * _D]
        acc = acc + jnp.maximum(sl + tb, 0.0) * mj
    out_ref[0, :, :] = acc * inv_mat


def kernel(ssg_rel, ssg_obj, ssg_att, W_emb, W_sbj, b_sbj, W_objr, b_objr,
           W_rel, b_rel, W_obj, b_obj, W_att, b_att):
    f32 = jnp.float32
    sbj_id = ssg_rel[:, :, 0]
    obj_id = ssg_rel[:, :, 1]
    pred_id = ssg_rel[:, :, 2]

    idx_a = jnp.concatenate([pred_id.reshape(-1), ssg_obj.reshape(-1)])
    ga = _sc_gather(W_emb, idx_a.reshape(_NW, -1, _CHUNK),
                    _ROWS_A // (_NW * _CHUNK))
    att_t = jnp.transpose(ssg_att, (0, 2, 1))
    idx_b = att_t.reshape(-1)
    gb = _sc_gather(W_emb, idx_b.reshape(_NW, -1, _CHUNK),
                    _ROWS_B // (_NW * _CHUNK))

    def b3(b):
        return (b, 0, 0)

    def w2(b):
        return (0, 0)

    out_shape = jax.ShapeDtypeStruct((_B, _NOBJ + _NREL + _NOBJ, _D), f32)

    feat_a = pl.pallas_call(
        _tc_a_body,
        grid=(_B,),
        in_specs=[
            pl.BlockSpec((_NOBJ, _D), lambda b: (_B * _NREL // _NOBJ + b, 0)),
            pl.BlockSpec((_NREL, _D), lambda b: (b, 0)),
            pl.BlockSpec((1, 1, _NREL), b3),
            pl.BlockSpec((1, 1, _NREL), b3),
            pl.BlockSpec((1, 1, _NREL), b3),
            pl.BlockSpec((1, 1, _NOBJ), b3),
            pl.BlockSpec((3 * _D, _D), w2),
            pl.BlockSpec((1, _D), w2),
            pl.BlockSpec((3 * _D, _D), w2),
            pl.BlockSpec((1, _D), w2),
            pl.BlockSpec((3 * _D, _D), w2),
            pl.BlockSpec((1, _D), w2),
            pl.BlockSpec((_D, _D), w2),
            pl.BlockSpec((1, _D), w2),
        ],
        out_specs=pl.BlockSpec((1, _NOBJ + _NREL + _NOBJ, _D), b3),
        out_shape=out_shape,
    )(ga, ga,
      sbj_id[:, None, :], obj_id[:, None, :], pred_id[:, None, :],
      ssg_obj[:, None, :],
      W_sbj, b_sbj[None, :], W_objr, b_objr[None, :],
      W_rel, b_rel[None, :], W_obj, b_obj[None, :])

    ssg_feat = pl.pallas_call(
        _tc_b_body,
        grid=(_B,),
        in_specs=[
            pl.BlockSpec(memory_space=pl.ANY),
            pl.BlockSpec((_NOBJ, _D), lambda b: (_B * _NREL // _NOBJ + b, 0)),
            pl.BlockSpec((_NATT * _NOBJ, _D), lambda b: (b, 0)),
            pl.BlockSpec((1, _NATT, _NOBJ), b3),
            pl.BlockSpec((2 * _D, _D), w2),
            pl.BlockSpec((1, _D), w2),
        ],
        out_specs=pl.BlockSpec((1, _NOBJ, _D), lambda b: (b, 3, 0)),
        out_shape=out_shape,
        input_output_aliases={0: 0},
    )(feat_a, ga, gb, att_t, W_att, b_att[None, :])

    cnt = jnp.sum(ssg_att != 1, axis=-1)
    ssg_mask = jnp.concatenate(
        [ssg_obj == 1, pred_id == 1, cnt == 0], axis=1)[:, None, None, :]
    return ssg_feat, ssg_mask


# speedup vs baseline: 591.2369x; 1.0308x over previous
import functools

import jax
import jax.numpy as jnp
from jax import lax
from jax.experimental import pallas as pl
from jax.experimental.pallas import tpu as pltpu
from jax.experimental.pallas import tpu_sc as plsc

_B = 32
_NREL = 1024
_NOBJ = 512
_NATT = 8
_D = 128

_NW = 32
_CHUNK = 128
_K = 4

_ROWS_A = _B * (_NREL + _NOBJ)
_ROWS_B = _B * _NOBJ * _NATT


_GC = 2


def _sc_gather(table, idx_w, nchunks):
    per_w = nchunks * _CHUNK
    groups = nchunks // _GC
    grp_rows = _GC * _CHUNK
    mesh = plsc.VectorSubcoreMesh(core_axis_name="c", subcore_axis_name="s")

    @functools.partial(
        pl.kernel,
        out_type=jax.ShapeDtypeStruct((_NW * per_w, _D), jnp.float32),
        mesh=mesh,
        scratch_types=[
            pltpu.VMEM((nchunks, _CHUNK), jnp.int32),
            pltpu.VMEM((grp_rows, _D), jnp.float32),
            pltpu.VMEM((grp_rows, _D), jnp.float32),
            pltpu.SemaphoreType.DMA,
            pltpu.SemaphoreType.DMA,
            pltpu.SemaphoreType.DMA,
            pltpu.SemaphoreType.DMA,
        ],
    )
    def gather_kernel(table_hbm, idx_hbm, out_hbm, idx_v,
                      rows_a, rows_b, gsa, gsb, wsa, wsb):
        wid = lax.axis_index("s") * 2 + lax.axis_index("c")
        base = wid * per_w
        pltpu.sync_copy(idx_hbm.at[wid], idx_v)

        def fire_g(buf, gsem, grp):
            return [pltpu.async_copy(
                table_hbm.at[idx_v.at[grp * _GC + j]],
                buf.at[pl.ds(j * _CHUNK, _CHUNK)], gsem)
                for j in range(_GC)]

        def fire_w(buf, wsem, grp):
            return pltpu.async_copy(
                buf, out_hbm.at[pl.ds(base + grp * grp_rows, grp_rows)], wsem)

        def drain(buf, sem):
            pltpu.make_async_copy(
                out_hbm.at[pl.ds(base, grp_rows)], buf, sem).wait()

        fire_g(rows_a, gsa, 0)

        @pl.loop(0, groups, step=2)
        def _pair(k):
            drain(rows_a, gsa)
            wa = fire_w(rows_a, wsa, k)

            @pl.when(k > 0)
            def _():
                drain(rows_b, wsb)

            gb = fire_g(rows_b, gsb, k + 1)

            for d in gb:
                d.wait()
            fire_w(rows_b, wsb, k + 1)

            @pl.when(k + 2 < groups)
            def _():
                wa.wait()
                fire_g(rows_a, gsa, k + 2)

        drain(rows_a, wsa)
        drain(rows_b, wsb)

    return gather_kernel(table, idx_w)


def _tc_a_body(obj_emb_ref, rel_emb_ref,
               ids_row_ref, pred_row_ref, objid_row_ref,
               w_sbj_ref, b_sbj_ref, w_objr_ref, b_objr_ref,
               w_rel_ref, b_rel_ref, w_obj_ref, b_obj_ref, out_ref):
    f32 = jnp.float32
    obj_emb = obj_emb_ref[...]
    rel_emb = rel_emb_ref[...]

    ids_r = ids_row_ref[0]
    pred_r = pred_row_ref[0]

    keep_rel = jnp.transpose((pred_r != 1).astype(f32))
    keep_obj = jnp.transpose((objid_row_ref[0] != 1).astype(f32))

    ct = lax.broadcasted_iota(jnp.int32, (_NOBJ, 2 * _NREL), 0)
    so_oh_t = (ct == ids_r).astype(f32)

    dn_t = (((0,), (0,)), ((), ()))
    both_feat = lax.dot_general(so_oh_t, obj_emb, dn_t,
                                preferred_element_type=f32)
    rel_sbj_feat = both_feat[0:_NREL, :]
    rel_obj_feat = both_feat[_NREL:, :]

    def mlp3(x1, x2, x3, w_ref, b_ref):
        x = jnp.concatenate([x1, x2, x3], axis=1)
        acc = jnp.dot(x, w_ref[...], preferred_element_type=f32)
        return jnp.maximum(acc + b_ref[...], 0.0)

    rel_sbj_new = mlp3(rel_sbj_feat, rel_obj_feat, rel_emb,
                       w_sbj_ref, b_sbj_ref) * keep_rel
    rel_obj_new = mlp3(rel_sbj_new, rel_obj_feat, rel_emb,
                       w_objr_ref, b_objr_ref) * keep_rel
    rel_rel_feat = mlp3(rel_sbj_new, rel_obj_new, rel_emb,
                        w_rel_ref, b_rel_ref) * keep_rel

    obj_feat = jnp.maximum(
        jnp.dot(obj_emb, w_obj_ref[...], preferred_element_type=f32)
        + b_obj_ref[...], 0.0) * keep_obj
    both_new = jnp.concatenate([rel_sbj_new, rel_obj_new], axis=0)
    obj_feat = obj_feat + jnp.dot(so_oh_t, both_new,
                                  preferred_element_type=f32)

    out_ref[0, 0:_NOBJ, :] = obj_feat * (1.0 / (2 * _NREL))
    out_ref[0, _NOBJ:_NOBJ + _NREL, :] = rel_rel_feat


def _tc_b_body(alias_ref, obj_emb_ref, att_emb_ref, attid_ref,
               w_att_ref, b_att_ref, out_ref):
    del alias_ref
    f32 = jnp.float32
    obj_emb = obj_emb_ref[...]
    att_emb = att_emb_ref[...]

    m8 = (attid_ref[0] != 1).astype(f32)
    cnt_row = jnp.sum(m8, axis=0, keepdims=True)
    inv_row = jnp.where(cnt_row == 0.0, 0.0,
                        1.0 / jnp.where(cnt_row == 0.0, 1.0, cnt_row))
    inv_mat = jnp.dot(jnp.transpose(inv_row), jnp.ones((1, _D), f32),
                      preferred_element_type=f32)
    m8t = jnp.transpose(m8)
    bd_r = lax.broadcasted_iota(jnp.int32, (_NATT, _NATT * _D), 0)
    bd_c = lax.broadcasted_iota(jnp.int32, (_NATT, _NATT * _D), 1)
    blockdiag = (bd_r == bd_c // _D).astype(f32)
    m_mat = jnp.dot(m8t, blockdiag,
                    preferred_element_type=f32)

    tb = jnp.dot(obj_emb, w_att_ref[0:_D],
                 preferred_element_type=f32) + b_att_ref[...]
    t_att = jnp.dot(att_emb, w_att_ref[_D:2 * _D],
                    preferred_element_type=f32)

    acc = jnp.zeros((_NOBJ, _D), f32)
    for j in range(_NATT):
        sl = t_att[j * _NOBJ:(j + 1) * _NOBJ, :]
        mj = m_mat[:, j * _D:(j + 1) * _D]
        acc = acc + jnp.maximum(sl + tb, 0.0) * mj
    out_ref[0, :, :] = acc * inv_mat


def kernel(ssg_rel, ssg_obj, ssg_att, W_emb, W_sbj, b_sbj, W_objr, b_objr,
           W_rel, b_rel, W_obj, b_obj, W_att, b_att):
    f32 = jnp.float32
    sbj_id = ssg_rel[:, :, 0]
    obj_id = ssg_rel[:, :, 1]
    pred_id = ssg_rel[:, :, 2]

    idx_a = jnp.concatenate([pred_id.reshape(-1), ssg_obj.reshape(-1)])
    ga = _sc_gather(W_emb, idx_a.reshape(_NW, -1, _CHUNK),
                    _ROWS_A // (_NW * _CHUNK))
    att_t = jnp.transpose(ssg_att, (0, 2, 1))
    idx_b = att_t.reshape(-1)
    gb = _sc_gather(W_emb, idx_b.reshape(_NW, -1, _CHUNK),
                    _ROWS_B // (_NW * _CHUNK))

    def b3(b):
        return (b, 0, 0)

    def w2(b):
        return (0, 0)

    out_shape = jax.ShapeDtypeStruct((_B, _NOBJ + _NREL + _NOBJ, _D), f32)

    feat_a = pl.pallas_call(
        _tc_a_body,
        grid=(_B,),
        in_specs=[
            pl.BlockSpec((_NOBJ, _D), lambda b: (_B * _NREL // _NOBJ + b, 0)),
            pl.BlockSpec((_NREL, _D), lambda b: (b, 0)),
            pl.BlockSpec((1, 1, 2 * _NREL), b3),
            pl.BlockSpec((1, 1, _NREL), b3),
            pl.BlockSpec((1, 1, _NOBJ), b3),
            pl.BlockSpec((3 * _D, _D), w2),
            pl.BlockSpec((1, _D), w2),
            pl.BlockSpec((3 * _D, _D), w2),
            pl.BlockSpec((1, _D), w2),
            pl.BlockSpec((3 * _D, _D), w2),
            pl.BlockSpec((1, _D), w2),
            pl.BlockSpec((_D, _D), w2),
            pl.BlockSpec((1, _D), w2),
        ],
        out_specs=pl.BlockSpec((1, _NOBJ + _NREL + _NOBJ, _D), b3),
        out_shape=out_shape,
    )(ga, ga,
      jnp.concatenate([sbj_id, obj_id], axis=1)[:, None, :],
      pred_id[:, None, :], ssg_obj[:, None, :],
      W_sbj, b_sbj[None, :], W_objr, b_objr[None, :],
      W_rel, b_rel[None, :], W_obj, b_obj[None, :])

    ssg_feat = pl.pallas_call(
        _tc_b_body,
        grid=(_B,),
        in_specs=[
            pl.BlockSpec(memory_space=pl.ANY),
            pl.BlockSpec((_NOBJ, _D), lambda b: (_B * _NREL // _NOBJ + b, 0)),
            pl.BlockSpec((_NATT * _NOBJ, _D), lambda b: (b, 0)),
            pl.BlockSpec((1, _NATT, _NOBJ), b3),
            pl.BlockSpec((2 * _D, _D), w2),
            pl.BlockSpec((1, _D), w2),
        ],
        out_specs=pl.BlockSpec((1, _NOBJ, _D), lambda b: (b, 3, 0)),
        out_shape=out_shape,
        input_output_aliases={0: 0},
    )(feat_a, ga, gb, att_t, W_att, b_att[None, :])

    cnt = jnp.sum(ssg_att != 1, axis=-1)
    ssg_mask = jnp.concatenate(
        [ssg_obj == 1, pred_id == 1, cnt == 0], axis=1)[:, None, None, :]
    return ssg_feat, ssg_mask


# speedup vs baseline: 632.4836x; 1.0698x over previous
import functools

import jax
import jax.numpy as jnp
from jax import lax
from jax.experimental import pallas as pl
from jax.experimental.pallas import tpu as pltpu
from jax.experimental.pallas import tpu_sc as plsc

_B = 32
_NREL = 1024
_NOBJ = 512
_NATT = 8
_D = 128

_NW = 32
_CHUNK = 128
_K = 4

_ROWS_A = _B * (_NREL + _NOBJ)
_ROWS_B = _B * _NOBJ * _NATT


_GC = 2


def _sc_gather(table, idx_w, nchunks):
    per_w = nchunks * _CHUNK
    groups = nchunks // _GC
    grp_rows = _GC * _CHUNK
    mesh = plsc.VectorSubcoreMesh(core_axis_name="c", subcore_axis_name="s")

    @functools.partial(
        pl.kernel,
        out_type=jax.ShapeDtypeStruct((_NW * per_w, _D), jnp.float32),
        mesh=mesh,
        scratch_types=[
            pltpu.VMEM((nchunks, _CHUNK), jnp.int32),
            pltpu.VMEM((grp_rows, _D), jnp.float32),
            pltpu.VMEM((grp_rows, _D), jnp.float32),
            pltpu.SemaphoreType.DMA,
            pltpu.SemaphoreType.DMA,
            pltpu.SemaphoreType.DMA,
            pltpu.SemaphoreType.DMA,
        ],
    )
    def gather_kernel(table_hbm, idx_hbm, out_hbm, idx_v,
                      rows_a, rows_b, gsa, gsb, wsa, wsb):
        wid = lax.axis_index("s") * 2 + lax.axis_index("c")
        base = wid * per_w
        pltpu.sync_copy(idx_hbm.at[wid], idx_v)

        def fire_g(buf, gsem, grp):
            return [pltpu.async_copy(
                table_hbm.at[idx_v.at[grp * _GC + j]],
                buf.at[pl.ds(j * _CHUNK, _CHUNK)], gsem)
                for j in range(_GC)]

        def fire_w(buf, wsem, grp):
            return pltpu.async_copy(
                buf, out_hbm.at[pl.ds(base + grp * grp_rows, grp_rows)], wsem)

        def drain(buf, sem):
            pltpu.make_async_copy(
                out_hbm.at[pl.ds(base, grp_rows)], buf, sem).wait()

        fire_g(rows_a, gsa, 0)

        @pl.loop(0, groups, step=2)
        def _pair(k):
            drain(rows_a, gsa)
            wa = fire_w(rows_a, wsa, k)

            @pl.when(k > 0)
            def _():
                drain(rows_b, wsb)

            gb = fire_g(rows_b, gsb, k + 1)

            for d in gb:
                d.wait()
            fire_w(rows_b, wsb, k + 1)

            @pl.when(k + 2 < groups)
            def _():
                wa.wait()
                fire_g(rows_a, gsa, k + 2)

        drain(rows_a, wsa)
        drain(rows_b, wsb)

    return gather_kernel(table, idx_w)


def _tc_a_body(obj_emb_ref, rel_emb_ref,
               ids_row_ref, pred_row_ref, objid_row_ref,
               w_sbj_ref, b_sbj_ref, w_objr_ref, b_objr_ref,
               w_rel_ref, b_rel_ref, w_obj_ref, b_obj_ref, out_ref):
  f32 = jnp.float32
  for p in range(2):
    obj_emb = obj_emb_ref[p * _NOBJ:(p + 1) * _NOBJ, :]
    rel_emb = rel_emb_ref[p * _NREL:(p + 1) * _NREL, :]

    ids_r = ids_row_ref[p]
    pred_r = pred_row_ref[p]

    keep_rel = jnp.transpose((pred_r != 1).astype(f32))
    keep_obj = jnp.transpose((objid_row_ref[p] != 1).astype(f32))

    ct = lax.broadcasted_iota(jnp.int32, (_NOBJ, 2 * _NREL), 0)
    so_oh_t = (ct == ids_r).astype(f32)

    dn_t = (((0,), (0,)), ((), ()))
    both_feat = lax.dot_general(so_oh_t, obj_emb, dn_t,
                                preferred_element_type=f32)
    rel_sbj_feat = both_feat[0:_NREL, :]
    rel_obj_feat = both_feat[_NREL:, :]

    def mlp3(x1, x2, x3, w_ref, b_ref):
        x = jnp.concatenate([x1, x2, x3], axis=1)
        acc = jnp.dot(x, w_ref[...], preferred_element_type=f32)
        return jnp.maximum(acc + b_ref[...], 0.0)

    rel_sbj_new = mlp3(rel_sbj_feat, rel_obj_feat, rel_emb,
                       w_sbj_ref, b_sbj_ref) * keep_rel
    rel_obj_new = mlp3(rel_sbj_new, rel_obj_feat, rel_emb,
                       w_objr_ref, b_objr_ref) * keep_rel
    rel_rel_feat = mlp3(rel_sbj_new, rel_obj_new, rel_emb,
                        w_rel_ref, b_rel_ref) * keep_rel

    obj_feat = jnp.maximum(
        jnp.dot(obj_emb, w_obj_ref[...], preferred_element_type=f32)
        + b_obj_ref[...], 0.0) * keep_obj
    both_new = jnp.concatenate([rel_sbj_new, rel_obj_new], axis=0)
    obj_feat = obj_feat + jnp.dot(so_oh_t, both_new,
                                  preferred_element_type=f32)

    out_ref[p, 0:_NOBJ, :] = obj_feat * (1.0 / (2 * _NREL))
    out_ref[p, _NOBJ:_NOBJ + _NREL, :] = rel_rel_feat


def _tc_b_body(alias_ref, obj_emb_ref, att_emb_ref, attid_ref,
               w_att_ref, b_att_ref, out_ref):
  del alias_ref
  f32 = jnp.float32
  for p in range(2):
    obj_emb = obj_emb_ref[p * _NOBJ:(p + 1) * _NOBJ, :]
    att_emb = att_emb_ref[p * _NATT * _NOBJ:(p + 1) * _NATT * _NOBJ, :]

    m8 = (attid_ref[p] != 1).astype(f32)
    cnt_row = jnp.sum(m8, axis=0, keepdims=True)
    inv_row = jnp.where(cnt_row == 0.0, 0.0,
                        1.0 / jnp.where(cnt_row == 0.0, 1.0, cnt_row))
    inv_mat = jnp.dot(jnp.transpose(inv_row), jnp.ones((1, _D), f32),
                      preferred_element_type=f32)
    m8t = jnp.transpose(m8)
    bd_r = lax.broadcasted_iota(jnp.int32, (_NATT, _NATT * _D), 0)
    bd_c = lax.broadcasted_iota(jnp.int32, (_NATT, _NATT * _D), 1)
    blockdiag = (bd_r == bd_c // _D).astype(f32)
    m_mat = jnp.dot(m8t, blockdiag,
                    preferred_element_type=f32)

    tb = jnp.dot(obj_emb, w_att_ref[0:_D],
                 preferred_element_type=f32) + b_att_ref[...]
    t_att = jnp.dot(att_emb, w_att_ref[_D:2 * _D],
                    preferred_element_type=f32)

    acc = jnp.zeros((_NOBJ, _D), f32)
    for j in range(_NATT):
        sl = t_att[j * _NOBJ:(j + 1) * _NOBJ, :]
        mj = m_mat[:, j * _D:(j + 1) * _D]
        acc = acc + jnp.maximum(sl + tb, 0.0) * mj
    out_ref[p, :, :] = acc * inv_mat


def kernel(ssg_rel, ssg_obj, ssg_att, W_emb, W_sbj, b_sbj, W_objr, b_objr,
           W_rel, b_rel, W_obj, b_obj, W_att, b_att):
    f32 = jnp.float32
    sbj_id = ssg_rel[:, :, 0]
    obj_id = ssg_rel[:, :, 1]
    pred_id = ssg_rel[:, :, 2]

    idx_a = jnp.concatenate([pred_id.reshape(-1), ssg_obj.reshape(-1)])
    ga = _sc_gather(W_emb, idx_a.reshape(_NW, -1, _CHUNK),
                    _ROWS_A // (_NW * _CHUNK))
    att_t = jnp.transpose(ssg_att, (0, 2, 1))
    idx_b = att_t.reshape(-1)
    gb = _sc_gather(W_emb, idx_b.reshape(_NW, -1, _CHUNK),
                    _ROWS_B // (_NW * _CHUNK))

    def b3(b):
        return (b, 0, 0)

    def w2(b):
        return (0, 0)

    out_shape = jax.ShapeDtypeStruct((_B, _NOBJ + _NREL + _NOBJ, _D), f32)

    feat_a = pl.pallas_call(
        _tc_a_body,
        grid=(_B // 2,),
        in_specs=[
            pl.BlockSpec((2 * _NOBJ, _D),
                         lambda b: (_B * _NREL // (2 * _NOBJ) + b, 0)),
            pl.BlockSpec((2 * _NREL, _D), lambda b: (b, 0)),
            pl.BlockSpec((2, 1, 2 * _NREL), b3),
            pl.BlockSpec((2, 1, _NREL), b3),
            pl.BlockSpec((2, 1, _NOBJ), b3),
            pl.BlockSpec((3 * _D, _D), w2),
            pl.BlockSpec((1, _D), w2),
            pl.BlockSpec((3 * _D, _D), w2),
            pl.BlockSpec((1, _D), w2),
            pl.BlockSpec((3 * _D, _D), w2),
            pl.BlockSpec((1, _D), w2),
            pl.BlockSpec((_D, _D), w2),
            pl.BlockSpec((1, _D), w2),
        ],
        out_specs=pl.BlockSpec((2, _NOBJ + _NREL + _NOBJ, _D), b3),
        out_shape=out_shape,
    )(ga, ga,
      jnp.concatenate([sbj_id, obj_id], axis=1)[:, None, :],
      pred_id[:, None, :], ssg_obj[:, None, :],
      W_sbj, b_sbj[None, :], W_objr, b_objr[None, :],
      W_rel, b_rel[None, :], W_obj, b_obj[None, :])

    ssg_feat = pl.pallas_call(
        _tc_b_body,
        grid=(_B // 2,),
        in_specs=[
            pl.BlockSpec(memory_space=pl.ANY),
            pl.BlockSpec((2 * _NOBJ, _D),
                         lambda b: (_B * _NREL // (2 * _NOBJ) + b, 0)),
            pl.BlockSpec((2 * _NATT * _NOBJ, _D), lambda b: (b, 0)),
            pl.BlockSpec((2, _NATT, _NOBJ), b3),
            pl.BlockSpec((2 * _D, _D), w2),
            pl.BlockSpec((1, _D), w2),
        ],
        out_specs=pl.BlockSpec((2, _NOBJ, _D), lambda b: (b, 3, 0)),
        out_shape=out_shape,
        input_output_aliases={0: 0},
    )(feat_a, ga, gb, att_t, W_att, b_att[None, :])

    cnt = jnp.sum(ssg_att != 1, axis=-1)
    ssg_mask = jnp.concatenate(
        [ssg_obj == 1, pred_id == 1, cnt == 0], axis=1)[:, None, None, :]
    return ssg_feat, ssg_mask


# speedup vs baseline: 646.8511x; 1.0227x over previous
import functools

import jax
import jax.numpy as jnp
from jax import lax
from jax.experimental import pallas as pl
from jax.experimental.pallas import tpu as pltpu
from jax.experimental.pallas import tpu_sc as plsc

_B = 32
_NREL = 1024
_NOBJ = 512
_NATT = 8
_D = 128

_NW = 32
_CHUNK = 128
_K = 4

_ROWS_A = _B * (_NREL + _NOBJ)
_ROWS_B = _B * _NOBJ * _NATT


_GC = 2


def _sc_gather(table, idx_w, nchunks):
    per_w = nchunks * _CHUNK
    groups = nchunks // _GC
    grp_rows = _GC * _CHUNK
    mesh = plsc.VectorSubcoreMesh(core_axis_name="c", subcore_axis_name="s")

    @functools.partial(
        pl.kernel,
        out_type=jax.ShapeDtypeStruct((_NW * per_w, _D), jnp.float32),
        mesh=mesh,
        scratch_types=[
            pltpu.VMEM((nchunks, _CHUNK), jnp.int32),
            pltpu.VMEM((grp_rows, _D), jnp.float32),
            pltpu.VMEM((grp_rows, _D), jnp.float32),
            pltpu.SemaphoreType.DMA,
            pltpu.SemaphoreType.DMA,
            pltpu.SemaphoreType.DMA,
            pltpu.SemaphoreType.DMA,
        ],
    )
    def gather_kernel(table_hbm, idx_hbm, out_hbm, idx_v,
                      rows_a, rows_b, gsa, gsb, wsa, wsb):
        wid = lax.axis_index("s") * 2 + lax.axis_index("c")
        base = wid * per_w
        pltpu.sync_copy(idx_hbm.at[wid], idx_v)

        def fire_g(buf, gsem, grp):
            return [pltpu.async_copy(
                table_hbm.at[idx_v.at[grp * _GC + j]],
                buf.at[pl.ds(j * _CHUNK, _CHUNK)], gsem)
                for j in range(_GC)]

        def fire_w(buf, wsem, grp):
            return pltpu.async_copy(
                buf, out_hbm.at[pl.ds(base + grp * grp_rows, grp_rows)], wsem)

        def drain(buf, sem):
            pltpu.make_async_copy(
                out_hbm.at[pl.ds(base, grp_rows)], buf, sem).wait()

        fire_g(rows_a, gsa, 0)

        @pl.loop(0, groups, step=2)
        def _pair(k):
            drain(rows_a, gsa)
            wa = fire_w(rows_a, wsa, k)

            @pl.when(k > 0)
            def _():
                drain(rows_b, wsb)

            gb = fire_g(rows_b, gsb, k + 1)

            for d in gb:
                d.wait()
            fire_w(rows_b, wsb, k + 1)

            @pl.when(k + 2 < groups)
            def _():
                wa.wait()
                fire_g(rows_a, gsa, k + 2)

        drain(rows_a, wsa)
        drain(rows_b, wsb)

    return gather_kernel(table, idx_w)


def _tc_a_body(obj_emb_ref, rel_emb_ref,
               ids_row_ref, pred_row_ref, objid_row_ref,
               w_sbj_ref, b_sbj_ref, w_objr_ref, b_objr_ref,
               w_rel_ref, b_rel_ref, w_obj_ref, b_obj_ref, out_ref):
  f32 = jnp.float32
  for p in range(4):
    obj_emb = obj_emb_ref[p * _NOBJ:(p + 1) * _NOBJ, :]
    rel_emb = rel_emb_ref[p * _NREL:(p + 1) * _NREL, :]

    ids_r = ids_row_ref[p]
    pred_r = pred_row_ref[p]

    keep_rel = jnp.transpose((pred_r != 1).astype(f32))
    keep_obj = jnp.transpose((objid_row_ref[p] != 1).astype(f32))

    ct = lax.broadcasted_iota(jnp.int32, (_NOBJ, 2 * _NREL), 0)
    so_oh_t = (ct == ids_r).astype(f32)

    dn_t = (((0,), (0,)), ((), ()))
    both_feat = lax.dot_general(so_oh_t, obj_emb, dn_t,
                                preferred_element_type=f32)
    rel_sbj_feat = both_feat[0:_NREL, :]
    rel_obj_feat = both_feat[_NREL:, :]

    def mlp3(x1, x2, x3, w_ref, b_ref):
        x = jnp.concatenate([x1, x2, x3], axis=1)
        acc = jnp.dot(x, w_ref[...], preferred_element_type=f32)
        return jnp.maximum(acc + b_ref[...], 0.0)

    rel_sbj_new = mlp3(rel_sbj_feat, rel_obj_feat, rel_emb,
                       w_sbj_ref, b_sbj_ref) * keep_rel
    rel_obj_new = mlp3(rel_sbj_new, rel_obj_feat, rel_emb,
                       w_objr_ref, b_objr_ref) * keep_rel
    rel_rel_feat = mlp3(rel_sbj_new, rel_obj_new, rel_emb,
                        w_rel_ref, b_rel_ref) * keep_rel

    obj_feat = jnp.maximum(
        jnp.dot(obj_emb, w_obj_ref[...], preferred_element_type=f32)
        + b_obj_ref[...], 0.0) * keep_obj
    both_new = jnp.concatenate([rel_sbj_new, rel_obj_new], axis=0)
    obj_feat = obj_feat + jnp.dot(so_oh_t, both_new,
                                  preferred_element_type=f32)

    out_ref[p, 0:_NOBJ, :] = obj_feat * (1.0 / (2 * _NREL))
    out_ref[p, _NOBJ:_NOBJ + _NREL, :] = rel_rel_feat


def _tc_b_body(alias_ref, obj_emb_ref, att_emb_ref, attid_ref,
               w_att_ref, b_att_ref, out_ref):
  del alias_ref
  f32 = jnp.float32
  for p in range(4):
    obj_emb = obj_emb_ref[p * _NOBJ:(p + 1) * _NOBJ, :]
    att_emb = att_emb_ref[p * _NATT * _NOBJ:(p + 1) * _NATT * _NOBJ, :]

    m8 = (attid_ref[p] != 1).astype(f32)
    cnt_row = jnp.sum(m8, axis=0, keepdims=True)
    inv_row = jnp.where(cnt_row == 0.0, 0.0,
                        1.0 / jnp.where(cnt_row == 0.0, 1.0, cnt_row))
    inv_mat = jnp.dot(jnp.transpose(inv_row), jnp.ones((1, _D), f32),
                      preferred_element_type=f32)
    m8t = jnp.transpose(m8)
    bd_r = lax.broadcasted_iota(jnp.int32, (_NATT, _NATT * _D), 0)
    bd_c = lax.broadcasted_iota(jnp.int32, (_NATT, _NATT * _D), 1)
    blockdiag = (bd_r == bd_c // _D).astype(f32)
    m_mat = jnp.dot(m8t, blockdiag,
                    preferred_element_type=f32)

    tb = jnp.dot(obj_emb, w_att_ref[0:_D],
                 preferred_element_type=f32) + b_att_ref[...]
    t_att = jnp.dot(att_emb, w_att_ref[_D:2 * _D],
                    preferred_element_type=f32)

    acc = jnp.zeros((_NOBJ, _D), f32)
    for j in range(_NATT):
        sl = t_att[j * _NOBJ:(j + 1) * _NOBJ, :]
        mj = m_mat[:, j * _D:(j + 1) * _D]
        acc = acc + jnp.maximum(sl + tb, 0.0) * mj
    out_ref[p, :, :] = acc * inv_mat


def kernel(ssg_rel, ssg_obj, ssg_att, W_emb, W_sbj, b_sbj, W_objr, b_objr,
           W_rel, b_rel, W_obj, b_obj, W_att, b_att):
    f32 = jnp.float32
    sbj_id = ssg_rel[:, :, 0]
    obj_id = ssg_rel[:, :, 1]
    pred_id = ssg_rel[:, :, 2]

    idx_a = jnp.concatenate([pred_id.reshape(-1), ssg_obj.reshape(-1)])
    ga = _sc_gather(W_emb, idx_a.reshape(_NW, -1, _CHUNK),
                    _ROWS_A // (_NW * _CHUNK))
    att_t = jnp.transpose(ssg_att, (0, 2, 1))
    idx_b = att_t.reshape(-1)
    gb = _sc_gather(W_emb, idx_b.reshape(_NW, -1, _CHUNK),
                    _ROWS_B // (_NW * _CHUNK))

    def b3(b):
        return (b, 0, 0)

    def w2(b):
        return (0, 0)

    out_shape = jax.ShapeDtypeStruct((_B, _NOBJ + _NREL + _NOBJ, _D), f32)

    feat_a = pl.pallas_call(
        _tc_a_body,
        grid=(_B // 4,),
        in_specs=[
            pl.BlockSpec((4 * _NOBJ, _D),
                         lambda b: (_B * _NREL // (4 * _NOBJ) + b, 0)),
            pl.BlockSpec((4 * _NREL, _D), lambda b: (b, 0)),
            pl.BlockSpec((4, 1, 2 * _NREL), b3),
            pl.BlockSpec((4, 1, _NREL), b3),
            pl.BlockSpec((4, 1, _NOBJ), b3),
            pl.BlockSpec((3 * _D, _D), w2),
            pl.BlockSpec((1, _D), w2),
            pl.BlockSpec((3 * _D, _D), w2),
            pl.BlockSpec((1, _D), w2),
            pl.BlockSpec((3 * _D, _D), w2),
            pl.BlockSpec((1, _D), w2),
            pl.BlockSpec((_D, _D), w2),
            pl.BlockSpec((1, _D), w2),
        ],
        out_specs=pl.BlockSpec((4, _NOBJ + _NREL + _NOBJ, _D), b3),
        out_shape=out_shape,
    )(ga, ga,
      jnp.concatenate([sbj_id, obj_id], axis=1)[:, None, :],
      pred_id[:, None, :], ssg_obj[:, None, :],
      W_sbj, b_sbj[None, :], W_objr, b_objr[None, :],
      W_rel, b_rel[None, :], W_obj, b_obj[None, :])

    ssg_feat = pl.pallas_call(
        _tc_b_body,
        grid=(_B // 4,),
        in_specs=[
            pl.BlockSpec(memory_space=pl.ANY),
            pl.BlockSpec((4 * _NOBJ, _D),
                         lambda b: (_B * _NREL // (4 * _NOBJ) + b, 0)),
            pl.BlockSpec((4 * _NATT * _NOBJ, _D), lambda b: (b, 0)),
            pl.BlockSpec((4, _NATT, _NOBJ), b3),
            pl.BlockSpec((2 * _D, _D), w2),
            pl.BlockSpec((1, _D), w2),
        ],
        out_specs=pl.BlockSpec((4, _NOBJ, _D), lambda b: (b, 3, 0)),
        out_shape=out_shape,
        input_output_aliases={0: 0},
    )(feat_a, ga, gb, att_t, W_att, b_att[None, :])

    cnt = jnp.sum(ssg_att != 1, axis=-1)
    ssg_mask = jnp.concatenate(
        [ssg_obj == 1, pred_id == 1, cnt == 0], axis=1)[:, None, None, :]
    return ssg_feat, ssg_mask
